# Initial kernel scaffold; baseline (speedup 1.0000x reference)
import jax
import jax.numpy as jnp
from jax.experimental import pallas as pl


def kernel(movies, table):
    raise NotImplementedError("write your pallas kernel here")


# speedup vs baseline: 73.3259x; 73.3259x over previous
import functools

import jax
import jax.numpy as jnp
from jax import lax
from jax.experimental import pallas as pl
from jax.experimental.pallas import tpu as pltpu
from jax.experimental.pallas import tpu_sc as plsc

VOCAB = 2000
EMBED_DIM = 19
N_ROWS = 16384
N_COLS = 50
N_IDX = N_ROWS * N_COLS
NW = 32
PER_W = N_IDX // NW
UNROLL = 4
STEPS = PER_W // (16 * UNROLL)


def _rowsum_body(table_ref, out_ref):
    out_ref[...] = jnp.sum(table_ref[...], axis=1, keepdims=True)


def _finalize_body(part_ref, out_ref):
    total = jnp.sum(part_ref[...], axis=(0, 1), keepdims=True)
    out_ref[...] = jax.nn.sigmoid(total * (1.0 / float(N_IDX * EMBED_DIM)))


def _sc_gather_sum(idx_flat, rowsum):
    mesh = plsc.VectorSubcoreMesh(core_axis_name="c", subcore_axis_name="s")

    @functools.partial(
        pl.kernel,
        mesh=mesh,
        compiler_params=pltpu.CompilerParams(needs_layout_passes=False),
        out_type=jax.ShapeDtypeStruct((NW, 16), jnp.float32),
        scratch_types=[
            pltpu.VMEM((PER_W,), jnp.int32),
            pltpu.VMEM((VOCAB,), jnp.float32),
            pltpu.VMEM((16,), jnp.float32),
        ],
    )
    def sc_kernel(idx_hbm, rs_hbm, out_hbm, idx_v, rs_v, acc_v):
        c = lax.axis_index("c")
        s = lax.axis_index("s")
        wid = s * 2 + c
        base = wid * PER_W
        pltpu.sync_copy(idx_hbm.at[pl.ds(base, PER_W)], idx_v)
        pltpu.sync_copy(rs_hbm, rs_v)

        def body(i, carry):
            accs = []
            for u in range(UNROLL):
                off = (i * UNROLL + u) * 16
                iv = idx_v[pl.ds(off, 16)]
                accs.append(carry[u] + plsc.load_gather(rs_v, [iv]))
            return tuple(accs)

        init = tuple(jnp.zeros((16,), jnp.float32) for _ in range(UNROLL))
        accs = lax.fori_loop(0, STEPS, body, init)
        total = accs[0]
        for u in range(1, UNROLL):
            total = total + accs[u]
        acc_v[...] = total
        pltpu.sync_copy(acc_v, out_hbm.at[wid])

    return sc_kernel(idx_flat, rowsum)


def kernel(movies, table):
    rowsum = pl.pallas_call(
        _rowsum_body,
        out_shape=jax.ShapeDtypeStruct((VOCAB, 1), jnp.float32),
    )(table)
    idx_flat = movies.reshape(N_IDX)
    partials = _sc_gather_sum(idx_flat, rowsum.reshape(VOCAB))
    out = pl.pallas_call(
        _finalize_body,
        out_shape=jax.ShapeDtypeStruct((1, 1), jnp.float32),
    )(partials)
    return out.reshape(())


# speedup vs baseline: 82.1626x; 1.1205x over previous
import functools

import jax
import jax.numpy as jnp
from jax import lax
from jax.experimental import pallas as pl
from jax.experimental.pallas import tpu as pltpu
from jax.experimental.pallas import tpu_sc as plsc

VOCAB = 2000
EMBED_DIM = 19
TFLAT = VOCAB * EMBED_DIM
N_IDX = 16384 * 50
NS = 16
PER_S = N_IDX // NS
RPS = 128
SLICE = RPS * EMBED_DIM
MAX_START = TFLAT - SLICE
UNROLL = 8
STEPS = PER_S // (16 * UNROLL)
INV_N = 1.0 / float(N_IDX * EMBED_DIM)


def _sc_embedding_mean_sigmoid(idx_flat, table_flat):
    mesh = plsc.VectorSubcoreMesh(
        core_axis_name="c", subcore_axis_name="s", num_cores=1
    )

    @functools.partial(
        pl.kernel,
        mesh=mesh,
        compiler_params=pltpu.CompilerParams(needs_layout_passes=False),
        out_type=jax.ShapeDtypeStruct((16,), jnp.float32),
        scratch_types=[
            pltpu.VMEM((PER_S,), jnp.int32),
            pltpu.VMEM((SLICE,), jnp.float32),
            pltpu.VMEM((RPS,), jnp.float32),
            pltpu.VMEM((NS * RPS,), jnp.float32),
            pltpu.VMEM((16,), jnp.float32),
            pltpu.VMEM((NS, 16), jnp.float32),
            pltpu.VMEM_SHARED((NS * RPS,), jnp.float32),
            pltpu.VMEM_SHARED((NS, 16), jnp.float32),
            pltpu.SemaphoreType.DMA,
        ],
    )
    def k(idx_hbm, tab_hbm, out_hbm, idx_v, tab_v, rs_local, rs_v, stage_v,
          part_v, sh_rs, sh_part, sem):
        sid = lax.axis_index("s")
        idx_cp = pltpu.async_copy(
            idx_hbm.at[pl.ds(sid * PER_S, PER_S)], idx_v, sem
        )

        start = pl.multiple_of(jnp.minimum(sid * SLICE, MAX_START), 8)
        pltpu.sync_copy(tab_hbm.at[pl.ds(start, SLICE)], tab_v)
        lanes = lax.iota(jnp.int32, 16)
        for c in range(RPS // 16):
            r = sid * RPS + c * 16 + lanes
            rc = jnp.minimum(r, VOCAB - 1)
            base = rc * EMBED_DIM - start
            acc = plsc.load_gather(tab_v, [base])
            for d in range(1, EMBED_DIM):
                acc = acc + plsc.load_gather(tab_v, [base + d])
            acc = jnp.where(r < VOCAB, acc, 0.0)
            rs_local[pl.ds(c * 16, 16)] = acc
        pltpu.sync_copy(rs_local, sh_rs.at[pl.ds(sid * RPS, RPS)])
        plsc.subcore_barrier()
        pltpu.sync_copy(sh_rs, rs_v)
        idx_cp.wait()

        def body(i, carry):
            out = []
            for u in range(UNROLL):
                off = (i * UNROLL + u) * 16
                iv = idx_v[pl.ds(off, 16)]
                out.append(carry[u] + plsc.load_gather(rs_v, [iv]))
            return tuple(out)

        accs = lax.fori_loop(
            0, STEPS, body,
            tuple(jnp.zeros((16,), jnp.float32) for _ in range(UNROLL)),
        )
        tot = accs[0]
        for u in range(1, UNROLL):
            tot = tot + accs[u]
        stage_v[...] = tot
        pltpu.sync_copy(stage_v, sh_part.at[sid])
        plsc.subcore_barrier()

        @pl.when(sid == 0)
        def _():
            pltpu.sync_copy(sh_part, part_v)
            tv = part_v[0]
            for i in range(1, NS):
                tv = tv + part_v[i]
            s = jnp.sum(tv) * INV_N
            vec = jnp.broadcast_to(s, (16,))
            stage_v[...] = 1.0 / (1.0 + jnp.exp(-vec))
            pltpu.sync_copy(stage_v, out_hbm)

    return k(idx_flat, table_flat)


def kernel(movies, table):
    out = _sc_embedding_mean_sigmoid(
        movies.reshape(N_IDX), table.reshape(TFLAT)
    )
    return out[0]
